# Initial kernel scaffold; baseline (speedup 1.0000x reference)
import jax
import jax.numpy as jnp
from jax.experimental import pallas as pl


def kernel(input_f, target, char_dic, ln1_gamma, ln1_beta):
    raise NotImplementedError("write your pallas kernel here")


# speedup vs baseline: 1.2169x; 1.2169x over previous
import functools

import jax
import jax.numpy as jnp
from jax.experimental import pallas as pl
from jax.experimental.pallas import tpu as pltpu

_EPS = 1e-5


def _ln_kernel(x_ref, g_ref, b_ref, o_ref):
    x = x_ref[...]
    mu = jnp.mean(x, axis=-1, keepdims=True)
    var = jnp.mean((x - mu) ** 2, axis=-1, keepdims=True)
    o_ref[...] = (x - mu) * jax.lax.rsqrt(var + _EPS) * g_ref[...] + b_ref[...]


def _pos_kernel(nr, nc, lt_r_ref, lt_c_ref, lab_r_ref, lab_c_ref, o_ref):
    i = pl.program_id(0)
    j = pl.program_id(1)

    @pl.when(jnp.logical_and(i == 0, j == 0))
    def _init():
        o_ref[...] = jnp.zeros_like(o_ref)

    s = jax.lax.dot_general(
        lt_r_ref[...], lt_c_ref[...],
        (((1,), (1,)), ((), ())),
        preferred_element_type=jnp.float32,
    ) * (1.0 / 768.0)
    lab_r = lab_r_ref[0, 0, :]
    lab_c = lab_c_ref[0, 0, :]
    m = lab_r[:, None] == lab_c[None, :]
    o_ref[...] += jnp.sum(jnp.where(m, jnp.exp(s), 0.0)).reshape(1, 1)


def _seg_kernel(n_cls, lt_ref, lab_ref, dic_ref, seg_ref, cnt_ref, dpos_ref):
    i = pl.program_id(0)

    @pl.when(i == 0)
    def _init():
        seg_ref[...] = jnp.zeros_like(seg_ref)
        cnt_ref[...] = jnp.zeros_like(cnt_ref)
        dpos_ref[...] = jnp.zeros_like(dpos_ref)

    lab = lab_ref[0, 0, :]
    t = lt_ref[...]
    cls = jax.lax.broadcasted_iota(jnp.int32, (n_cls, lab.shape[0]), 0)
    onehot = (cls == lab[None, :]).astype(jnp.float32)
    seg_ref[...] += jax.lax.dot_general(
        onehot, t, (((1,), (0,)), ((), ())),
        preferred_element_type=jnp.float32,
    )
    cnt_ref[...] += jnp.sum(onehot, axis=1)[None, :]
    g = jax.lax.dot_general(
        dic_ref[...], t, (((1,), (1,)), ((), ())),
        preferred_element_type=jnp.float32,
    ) * (1.0 / 768.0)
    dpos_ref[...] += (2.0 * jnp.sum(jnp.where(onehot > 0.0, jnp.exp(g), 0.0))).reshape(1, 1)


def _final_kernel(n_cls, dic_ref, seg_ref, cnt_ref, g_ref, b_ref,
                  posT_ref, dpos_ref, o_ref):
    dic = dic_ref[...]
    diag = jnp.sum(dic * dic, axis=1) * (1.0 / 768.0)
    pos_dic_diag = jnp.sum(jnp.exp(diag))

    cnt = cnt_ref[0, :]
    char_tem = (dic + seg_ref[...]) / (1.0 + cnt)[:, None]
    updated = dic + 0.1 * char_tem
    row = jax.lax.broadcasted_iota(jnp.int32, (n_cls, 1), 0)
    new_dic = jnp.where(row == 0, dic, updated)
    mu = jnp.mean(new_dic, axis=-1, keepdims=True)
    var = jnp.mean((new_dic - mu) ** 2, axis=-1, keepdims=True)
    nd = (new_dic - mu) * jax.lax.rsqrt(var + _EPS) * g_ref[...] + b_ref[...]

    p = jax.lax.dot_general(
        nd, nd, (((1,), (1,)), ((), ())),
        preferred_element_type=jnp.float32,
    ) * (1.0 / 768.0)
    rr = jax.lax.broadcasted_iota(jnp.int32, (n_cls, n_cls), 0)
    cc = jax.lax.broadcasted_iota(jnp.int32, (n_cls, n_cls), 1)
    keep = jnp.logical_and(rr > 0, cc > 0)
    neg = jnp.sum(jnp.where(keep, jnp.exp(p), 0.0))

    pos = jnp.sum(posT_ref[...]) + jnp.sum(dpos_ref[...]) + pos_dic_diag
    o_ref[...] = (-jnp.log(pos / neg)).reshape(1, 1)


def kernel(input_f, target, char_dic, ln1_gamma, ln1_beta):
    B, L, D = input_f.shape
    N = B * L
    n_cls = char_dic.shape[0]
    tokens = input_f.reshape(N, D)
    labels = target.reshape(N).astype(jnp.int32)
    g2 = ln1_gamma.reshape(1, D)
    b2 = ln1_beta.reshape(1, D)

    TLN = 512
    ln_tok = pl.pallas_call(
        _ln_kernel,
        grid=(N // TLN,),
        in_specs=[
            pl.BlockSpec((TLN, D), lambda i: (i, 0)),
            pl.BlockSpec((1, D), lambda i: (0, 0)),
            pl.BlockSpec((1, D), lambda i: (0, 0)),
        ],
        out_specs=pl.BlockSpec((TLN, D), lambda i: (i, 0)),
        out_shape=jax.ShapeDtypeStruct((N, D), jnp.float32),
    )(tokens, g2, b2)

    T = 512
    nt = N // T
    lab3 = labels.reshape(nt, 1, T)

    posT = pl.pallas_call(
        functools.partial(_pos_kernel, nt, nt),
        grid=(nt, nt),
        in_specs=[
            pl.BlockSpec((T, D), lambda i, j: (i, 0)),
            pl.BlockSpec((T, D), lambda i, j: (j, 0)),
            pl.BlockSpec((1, 1, T), lambda i, j: (i, 0, 0)),
            pl.BlockSpec((1, 1, T), lambda i, j: (j, 0, 0)),
        ],
        out_specs=pl.BlockSpec((1, 1), lambda i, j: (0, 0)),
        out_shape=jax.ShapeDtypeStruct((1, 1), jnp.float32),
    )(ln_tok, ln_tok, lab3, lab3)

    seg, cnt, dpos = pl.pallas_call(
        functools.partial(_seg_kernel, n_cls),
        grid=(nt,),
        in_specs=[
            pl.BlockSpec((T, D), lambda i: (i, 0)),
            pl.BlockSpec((1, 1, T), lambda i: (i, 0, 0)),
            pl.BlockSpec((n_cls, D), lambda i: (0, 0)),
        ],
        out_specs=[
            pl.BlockSpec((n_cls, D), lambda i: (0, 0)),
            pl.BlockSpec((1, n_cls), lambda i: (0, 0)),
            pl.BlockSpec((1, 1), lambda i: (0, 0)),
        ],
        out_shape=[
            jax.ShapeDtypeStruct((n_cls, D), jnp.float32),
            jax.ShapeDtypeStruct((1, n_cls), jnp.float32),
            jax.ShapeDtypeStruct((1, 1), jnp.float32),
        ],
    )(ln_tok, lab3, char_dic)

    loss = pl.pallas_call(
        functools.partial(_final_kernel, n_cls),
        in_specs=[
            pl.BlockSpec((n_cls, D), lambda: (0, 0)),
            pl.BlockSpec((n_cls, D), lambda: (0, 0)),
            pl.BlockSpec((1, n_cls), lambda: (0, 0)),
            pl.BlockSpec((1, D), lambda: (0, 0)),
            pl.BlockSpec((1, D), lambda: (0, 0)),
            pl.BlockSpec((1, 1), lambda: (0, 0)),
            pl.BlockSpec((1, 1), lambda: (0, 0)),
        ],
        out_specs=pl.BlockSpec((1, 1), lambda: (0, 0)),
        out_shape=jax.ShapeDtypeStruct((1, 1), jnp.float32),
    )(char_dic, seg, cnt, g2, b2, posT, dpos)

    return loss.reshape(1)


# speedup vs baseline: 1.4818x; 1.2177x over previous
import functools

import jax
import jax.numpy as jnp
from jax.experimental import pallas as pl

_EPS = 1e-5


def _ln_kernel(x_ref, g_ref, b_ref, o_ref):
    x = x_ref[...]
    mu = jnp.mean(x, axis=-1, keepdims=True)
    var = jnp.mean((x - mu) ** 2, axis=-1, keepdims=True)
    o_ref[...] = (x - mu) * jax.lax.rsqrt(var + _EPS) * g_ref[...] + b_ref[...]


def _band_kernel(n_cls, T, lab_ref, lt_ref, dic_ref,
                 posT_ref, seg_ref, cnt_ref, dpos_ref):
    i = pl.program_id(0)

    @pl.when(i == 0)
    def _init():
        posT_ref[...] = jnp.zeros_like(posT_ref)
        seg_ref[...] = jnp.zeros_like(seg_ref)
        cnt_ref[...] = jnp.zeros_like(cnt_ref)
        dpos_ref[...] = jnp.zeros_like(dpos_ref)

    row = lt_ref[pl.ds(i * T, T), :]
    labr = lab_ref[i, 0, :]

    cls = jax.lax.broadcasted_iota(jnp.int32, (n_cls, T), 0)
    onehot = (cls == labr[None, :]).astype(jnp.float32)
    seg_ref[...] += jax.lax.dot_general(
        onehot, row, (((1,), (0,)), ((), ())),
        preferred_element_type=jnp.float32,
    )
    cnt_ref[...] += jnp.sum(onehot, axis=1)[None, :]
    g = jax.lax.dot_general(
        dic_ref[...], row, (((1,), (1,)), ((), ())),
        preferred_element_type=jnp.float32,
    ) * (1.0 / 768.0)
    dpos_ref[...] += (2.0 * jnp.sum(jnp.where(onehot > 0.0, jnp.exp(g), 0.0))).reshape(1, 1)

    lab_all = lab_ref[...]
    row_first = jnp.min(labr)
    row_last = jnp.max(labr)
    jlo = jnp.sum((lab_all < row_first).astype(jnp.int32)) // T
    jhi = (jnp.sum((lab_all <= row_last).astype(jnp.int32)) - 1) // T

    def body(j, acc):
        col = lt_ref[pl.ds(j * T, T), :]
        s = jax.lax.dot_general(
            row, col, (((1,), (1,)), ((), ())),
            preferred_element_type=jnp.float32,
        ) * (1.0 / 768.0)
        labc = lab_ref[j, 0, :]
        m = labr[:, None] == labc[None, :]
        return acc + jnp.sum(jnp.where(m, jnp.exp(s), 0.0))

    acc = jax.lax.fori_loop(jlo, jhi + 1, body, jnp.float32(0.0))
    posT_ref[...] += acc.reshape(1, 1)


def _final_kernel(n_cls, dic_ref, seg_ref, cnt_ref, g_ref, b_ref,
                  posT_ref, dpos_ref, o_ref):
    dic = dic_ref[...]
    diag = jnp.sum(dic * dic, axis=1) * (1.0 / 768.0)
    pos_dic_diag = jnp.sum(jnp.exp(diag))

    cnt = cnt_ref[0, :]
    char_tem = (dic + seg_ref[...]) / (1.0 + cnt)[:, None]
    updated = dic + 0.1 * char_tem
    row = jax.lax.broadcasted_iota(jnp.int32, (n_cls, 1), 0)
    new_dic = jnp.where(row == 0, dic, updated)
    mu = jnp.mean(new_dic, axis=-1, keepdims=True)
    var = jnp.mean((new_dic - mu) ** 2, axis=-1, keepdims=True)
    nd = (new_dic - mu) * jax.lax.rsqrt(var + _EPS) * g_ref[...] + b_ref[...]

    p = jax.lax.dot_general(
        nd, nd, (((1,), (1,)), ((), ())),
        preferred_element_type=jnp.float32,
    ) * (1.0 / 768.0)
    rr = jax.lax.broadcasted_iota(jnp.int32, (n_cls, n_cls), 0)
    cc = jax.lax.broadcasted_iota(jnp.int32, (n_cls, n_cls), 1)
    keep = jnp.logical_and(rr > 0, cc > 0)
    neg = jnp.sum(jnp.where(keep, jnp.exp(p), 0.0))

    pos = jnp.sum(posT_ref[...]) + jnp.sum(dpos_ref[...]) + pos_dic_diag
    o_ref[...] = (-jnp.log(pos / neg)).reshape(1, 1)


def kernel(input_f, target, char_dic, ln1_gamma, ln1_beta):
    B, L, D = input_f.shape
    N = B * L
    n_cls = char_dic.shape[0]
    tokens = input_f.reshape(N, D)
    labels = target.reshape(N).astype(jnp.int32)
    g2 = ln1_gamma.reshape(1, D)
    b2 = ln1_beta.reshape(1, D)

    perm = jnp.argsort(labels)
    slab = labels[perm]
    stok = tokens[perm]

    TLN = 512
    ln_tok = pl.pallas_call(
        _ln_kernel,
        grid=(N // TLN,),
        in_specs=[
            pl.BlockSpec((TLN, D), lambda i: (i, 0)),
            pl.BlockSpec((1, D), lambda i: (0, 0)),
            pl.BlockSpec((1, D), lambda i: (0, 0)),
        ],
        out_specs=pl.BlockSpec((TLN, D), lambda i: (i, 0)),
        out_shape=jax.ShapeDtypeStruct((N, D), jnp.float32),
    )(stok, g2, b2)

    T = 256
    nt = N // T
    lab3 = slab.reshape(nt, 1, T)

    posT, seg, cnt, dpos = pl.pallas_call(
        functools.partial(_band_kernel, n_cls, T),
        grid=(nt,),
        in_specs=[
            pl.BlockSpec((nt, 1, T), lambda i: (0, 0, 0)),
            pl.BlockSpec((N, D), lambda i: (0, 0)),
            pl.BlockSpec((n_cls, D), lambda i: (0, 0)),
        ],
        out_specs=[
            pl.BlockSpec((1, 1), lambda i: (0, 0)),
            pl.BlockSpec((n_cls, D), lambda i: (0, 0)),
            pl.BlockSpec((1, n_cls), lambda i: (0, 0)),
            pl.BlockSpec((1, 1), lambda i: (0, 0)),
        ],
        out_shape=[
            jax.ShapeDtypeStruct((1, 1), jnp.float32),
            jax.ShapeDtypeStruct((n_cls, D), jnp.float32),
            jax.ShapeDtypeStruct((1, n_cls), jnp.float32),
            jax.ShapeDtypeStruct((1, 1), jnp.float32),
        ],
    )(lab3, ln_tok, char_dic)

    loss = pl.pallas_call(
        functools.partial(_final_kernel, n_cls),
        in_specs=[
            pl.BlockSpec((n_cls, D), lambda: (0, 0)),
            pl.BlockSpec((n_cls, D), lambda: (0, 0)),
            pl.BlockSpec((1, n_cls), lambda: (0, 0)),
            pl.BlockSpec((1, D), lambda: (0, 0)),
            pl.BlockSpec((1, D), lambda: (0, 0)),
            pl.BlockSpec((1, 1), lambda: (0, 0)),
            pl.BlockSpec((1, 1), lambda: (0, 0)),
        ],
        out_specs=pl.BlockSpec((1, 1), lambda: (0, 0)),
        out_shape=jax.ShapeDtypeStruct((1, 1), jnp.float32),
    )(char_dic, seg, cnt, g2, b2, posT, dpos)

    return loss.reshape(1)


# speedup vs baseline: 1.4884x; 1.0044x over previous
import functools

import jax
import jax.numpy as jnp
from jax.experimental import pallas as pl
from jax.experimental.pallas import tpu as pltpu

_EPS = 1e-5


def _ln_rows(x, g, b):
    mu = jnp.mean(x, axis=-1, keepdims=True)
    var = jnp.mean((x - mu) ** 2, axis=-1, keepdims=True)
    return (x - mu) * jax.lax.rsqrt(var + _EPS) * g + b


def _mega_kernel(n_cls, T, nt, TLN, nln,
                 tok_ref, lab_ref, dic_ref, g_ref, b_ref, o_ref,
                 ln_ref, seg_ref, cnt_ref, posT_ref, dpos_ref):
    p = pl.program_id(0)

    @pl.when(p == 0)
    def _init():
        seg_ref[...] = jnp.zeros_like(seg_ref)
        cnt_ref[...] = jnp.zeros_like(cnt_ref)
        posT_ref[0, 0] = 0.0
        dpos_ref[0, 0] = 0.0

    @pl.when(p < nln)
    def _ln_phase():
        x = tok_ref[pl.ds(p * TLN, TLN), :]
        ln_ref[pl.ds(p * TLN, TLN), :] = _ln_rows(x, g_ref[...], b_ref[...])

    @pl.when(jnp.logical_and(p >= nln, p < nln + nt))
    def _band_phase():
        i = p - nln
        row = ln_ref[pl.ds(i * T, T), :]
        labr = lab_ref[i, 0, :]

        cls = jax.lax.broadcasted_iota(jnp.int32, (n_cls, T), 0)
        onehot = (cls == labr[None, :]).astype(jnp.float32)
        seg_ref[...] += jax.lax.dot_general(
            onehot, row, (((1,), (0,)), ((), ())),
            preferred_element_type=jnp.float32,
        )
        cnt_ref[...] += jnp.sum(onehot, axis=1)[None, :]
        g = jax.lax.dot_general(
            dic_ref[...], row, (((1,), (1,)), ((), ())),
            preferred_element_type=jnp.float32,
        ) * (1.0 / 768.0)
        dpos_ref[0, 0] += 2.0 * jnp.sum(jnp.where(onehot > 0.0, jnp.exp(g), 0.0))

        lab_all = lab_ref[...]
        row_first = jnp.min(labr)
        row_last = jnp.max(labr)
        jlo = jnp.sum((lab_all < row_first).astype(jnp.int32)) // T
        jhi = (jnp.sum((lab_all <= row_last).astype(jnp.int32)) - 1) // T

        def body(j, acc):
            col = ln_ref[pl.ds(j * T, T), :]
            s = jax.lax.dot_general(
                row, col, (((1,), (1,)), ((), ())),
                preferred_element_type=jnp.float32,
            ) * (1.0 / 768.0)
            labc = lab_ref[j, 0, :]
            m = labr[:, None] == labc[None, :]
            return acc + jnp.sum(jnp.where(m, jnp.exp(s), 0.0))

        acc = jax.lax.fori_loop(jlo, jhi + 1, body, jnp.float32(0.0))
        posT_ref[0, 0] += acc

    @pl.when(p == nln + nt)
    def _final_phase():
        dic = dic_ref[...]
        diag = jnp.sum(dic * dic, axis=1) * (1.0 / 768.0)
        pos_dic_diag = jnp.sum(jnp.exp(diag))

        cnt = cnt_ref[0, :]
        char_tem = (dic + seg_ref[...]) / (1.0 + cnt)[:, None]
        updated = dic + 0.1 * char_tem
        rowi = jax.lax.broadcasted_iota(jnp.int32, (n_cls, 1), 0)
        new_dic = jnp.where(rowi == 0, dic, updated)
        nd = _ln_rows(new_dic, g_ref[...], b_ref[...])

        sim = jax.lax.dot_general(
            nd, nd, (((1,), (1,)), ((), ())),
            preferred_element_type=jnp.float32,
        ) * (1.0 / 768.0)
        rr = jax.lax.broadcasted_iota(jnp.int32, (n_cls, n_cls), 0)
        cc = jax.lax.broadcasted_iota(jnp.int32, (n_cls, n_cls), 1)
        keep = jnp.logical_and(rr > 0, cc > 0)
        neg = jnp.sum(jnp.where(keep, jnp.exp(sim), 0.0))

        pos = posT_ref[0, 0] + dpos_ref[0, 0] + pos_dic_diag
        o_ref[...] = (-jnp.log(pos / neg)).reshape(1, 1)


def kernel(input_f, target, char_dic, ln1_gamma, ln1_beta):
    B, L, D = input_f.shape
    N = B * L
    n_cls = char_dic.shape[0]
    tokens = input_f.reshape(N, D)
    labels = target.reshape(N).astype(jnp.int32)
    g2 = ln1_gamma.reshape(1, D)
    b2 = ln1_beta.reshape(1, D)

    idx = jnp.arange(N, dtype=jnp.int32)
    skey = jnp.sort(labels * N + idx)
    perm = jnp.bitwise_and(skey, N - 1)
    slab = jax.lax.shift_right_logical(skey, 12)
    stok = jnp.take(tokens, perm, axis=0)

    T = 256
    nt = N // T
    TLN = 512
    nln = N // TLN
    lab3 = slab.reshape(nt, 1, T)

    loss = pl.pallas_call(
        functools.partial(_mega_kernel, n_cls, T, nt, TLN, nln),
        grid=(nln + nt + 1,),
        in_specs=[
            pl.BlockSpec((N, D), lambda p: (0, 0)),
            pl.BlockSpec((nt, 1, T), lambda p: (0, 0, 0)),
            pl.BlockSpec((n_cls, D), lambda p: (0, 0)),
            pl.BlockSpec((1, D), lambda p: (0, 0)),
            pl.BlockSpec((1, D), lambda p: (0, 0)),
        ],
        out_specs=pl.BlockSpec((1, 1), lambda p: (0, 0)),
        out_shape=jax.ShapeDtypeStruct((1, 1), jnp.float32),
        scratch_shapes=[
            pltpu.VMEM((N, D), jnp.float32),
            pltpu.VMEM((n_cls, D), jnp.float32),
            pltpu.VMEM((1, n_cls), jnp.float32),
            pltpu.SMEM((1, 1), jnp.float32),
            pltpu.SMEM((1, 1), jnp.float32),
        ],
    )(stok, lab3, char_dic, g2, b2)

    return loss.reshape(1)


# speedup vs baseline: 1.6649x; 1.1186x over previous
import functools

import jax
import jax.numpy as jnp
from jax.experimental import pallas as pl
from jax.experimental.pallas import tpu as pltpu

_EPS = 1e-5


def _ln_rows(x, g, b):
    mu = jnp.mean(x, axis=-1, keepdims=True)
    var = jnp.mean((x - mu) ** 2, axis=-1, keepdims=True)
    return (x - mu) * jax.lax.rsqrt(var + _EPS) * g + b


def _mega_kernel(n_cls, T, nt,
                 tok_ref, lab_ref, dic_ref, g_ref, b_ref, o_ref,
                 ln_ref, seg_ref, cnt_ref, posT_ref, dpos_ref):
    p = pl.program_id(0)

    @pl.when(p == 0)
    def _init():
        seg_ref[...] = jnp.zeros_like(seg_ref)
        cnt_ref[...] = jnp.zeros_like(cnt_ref)
        posT_ref[0, 0] = 0.0
        dpos_ref[0, 0] = 0.0

    @pl.when(p < nt)
    def _band_phase():
        i = p
        row = _ln_rows(tok_ref[pl.ds(i * T, T), :], g_ref[...], b_ref[...])
        ln_ref[pl.ds(i * T, T), :] = row
        labr = lab_ref[i, 0, :]

        cls = jax.lax.broadcasted_iota(jnp.int32, (n_cls, T), 0)
        onehot = (cls == labr[None, :]).astype(jnp.float32)
        seg_ref[...] += jax.lax.dot_general(
            onehot, row, (((1,), (0,)), ((), ())),
            preferred_element_type=jnp.float32,
        )
        cnt_ref[...] += jnp.sum(onehot, axis=1)[None, :]
        g = jax.lax.dot_general(
            dic_ref[...], row, (((1,), (1,)), ((), ())),
            preferred_element_type=jnp.float32,
        ) * (1.0 / 768.0)
        dpos_ref[0, 0] += 2.0 * jnp.sum(jnp.where(onehot > 0.0, jnp.exp(g), 0.0))

        lab_all = lab_ref[...]
        row_first = jnp.min(labr)
        jlo = jnp.sum((lab_all < row_first).astype(jnp.int32)) // T

        def body(j, acc):
            col = ln_ref[pl.ds(j * T, T), :]
            s = jax.lax.dot_general(
                row, col, (((1,), (1,)), ((), ())),
                preferred_element_type=jnp.float32,
            ) * (1.0 / 768.0)
            labc = lab_ref[j, 0, :]
            m = labr[:, None] == labc[None, :]
            w = jnp.where(j == i, 1.0, 2.0)
            return acc + w * jnp.sum(jnp.where(m, jnp.exp(s), 0.0))

        acc = jax.lax.fori_loop(jlo, i + 1, body, jnp.float32(0.0))
        posT_ref[0, 0] += acc

    @pl.when(p == nt)
    def _final_phase():
        dic = dic_ref[...]
        diag = jnp.sum(dic * dic, axis=1) * (1.0 / 768.0)
        pos_dic_diag = jnp.sum(jnp.exp(diag))

        cnt = cnt_ref[0, :]
        char_tem = (dic + seg_ref[...]) / (1.0 + cnt)[:, None]
        updated = dic + 0.1 * char_tem
        rowi = jax.lax.broadcasted_iota(jnp.int32, (n_cls, 1), 0)
        new_dic = jnp.where(rowi == 0, dic, updated)
        nd = _ln_rows(new_dic, g_ref[...], b_ref[...])

        sim = jax.lax.dot_general(
            nd, nd, (((1,), (1,)), ((), ())),
            preferred_element_type=jnp.float32,
        ) * (1.0 / 768.0)
        rr = jax.lax.broadcasted_iota(jnp.int32, (n_cls, n_cls), 0)
        cc = jax.lax.broadcasted_iota(jnp.int32, (n_cls, n_cls), 1)
        keep = jnp.logical_and(rr > 0, cc > 0)
        neg = jnp.sum(jnp.where(keep, jnp.exp(sim), 0.0))

        pos = posT_ref[0, 0] + dpos_ref[0, 0] + pos_dic_diag
        o_ref[...] = (-jnp.log(pos / neg)).reshape(1, 1)


def kernel(input_f, target, char_dic, ln1_gamma, ln1_beta):
    B, L, D = input_f.shape
    N = B * L
    n_cls = char_dic.shape[0]
    tokens = input_f.reshape(N, D)
    labels = target.reshape(N).astype(jnp.int32)
    g2 = ln1_gamma.reshape(1, D)
    b2 = ln1_beta.reshape(1, D)

    idx = jnp.arange(N, dtype=jnp.int32)
    skey = jnp.sort(labels * N + idx)
    perm = jnp.bitwise_and(skey, N - 1)
    slab = jax.lax.shift_right_logical(skey, 12)
    stok = jnp.take(tokens, perm, axis=0)

    T = 256
    nt = N // T
    lab3 = slab.reshape(nt, 1, T)

    loss = pl.pallas_call(
        functools.partial(_mega_kernel, n_cls, T, nt),
        grid=(nt + 1,),
        in_specs=[
            pl.BlockSpec((N, D), lambda p: (0, 0)),
            pl.BlockSpec((nt, 1, T), lambda p: (0, 0, 0)),
            pl.BlockSpec((n_cls, D), lambda p: (0, 0)),
            pl.BlockSpec((1, D), lambda p: (0, 0)),
            pl.BlockSpec((1, D), lambda p: (0, 0)),
        ],
        out_specs=pl.BlockSpec((1, 1), lambda p: (0, 0)),
        out_shape=jax.ShapeDtypeStruct((1, 1), jnp.float32),
        scratch_shapes=[
            pltpu.VMEM((N, D), jnp.float32),
            pltpu.VMEM((n_cls, D), jnp.float32),
            pltpu.VMEM((1, n_cls), jnp.float32),
            pltpu.SMEM((1, 1), jnp.float32),
            pltpu.SMEM((1, 1), jnp.float32),
        ],
    )(stok, lab3, char_dic, g2, b2)

    return loss.reshape(1)


# speedup vs baseline: 1.9300x; 1.1593x over previous
import functools

import jax
import jax.numpy as jnp
from jax import lax
from jax.experimental import pallas as pl
from jax.experimental.pallas import tpu as pltpu
from jax.experimental.pallas import tpu_sc as plsc

_EPS = 1e-5


def _sc_gather_body(b_per_w, nc, table_hbm, idx_hbm, out_hbm, idx_v, rows_v, sem):
    wid = lax.axis_index("s") * nc + lax.axis_index("c")
    base = wid * b_per_w
    pltpu.sync_copy(idx_hbm.at[pl.ds(base, b_per_w)], idx_v)
    pltpu.async_copy(table_hbm.at[idx_v], rows_v, sem).wait()
    pltpu.sync_copy(rows_v, out_hbm.at[pl.ds(base, b_per_w)])


def _sc_gather(table, idx):
    N, D = table.shape
    info = plsc.get_sparse_core_info()
    nc, ns = info.num_cores, info.num_subcores
    b_per_w = N // (nc * ns)
    mesh = plsc.VectorSubcoreMesh(core_axis_name="c", subcore_axis_name="s")
    return pl.kernel(
        functools.partial(_sc_gather_body, b_per_w, nc),
        mesh=mesh,
        out_type=jax.ShapeDtypeStruct((N, D), jnp.float32),
        scratch_types=[
            pltpu.VMEM((b_per_w,), jnp.int32),
            pltpu.VMEM((b_per_w, D), jnp.float32),
            pltpu.SemaphoreType.DMA,
        ],
    )(table, idx)


def _ln_rows(x, g, b):
    mu = jnp.mean(x, axis=-1, keepdims=True)
    var = jnp.mean((x - mu) ** 2, axis=-1, keepdims=True)
    return (x - mu) * jax.lax.rsqrt(var + _EPS) * g + b


def _mega_kernel(n_cls, T, nt,
                 tok_ref, lab_ref, dic_ref, g_ref, b_ref, o_ref,
                 ln_ref, seg_ref, cnt_ref, posT_ref, dpos_ref):
    p = pl.program_id(0)

    @pl.when(p == 0)
    def _init():
        seg_ref[...] = jnp.zeros_like(seg_ref)
        cnt_ref[...] = jnp.zeros_like(cnt_ref)
        posT_ref[0, 0] = 0.0
        dpos_ref[0, 0] = 0.0

    @pl.when(p < nt)
    def _band_phase():
        i = p
        row = _ln_rows(tok_ref[pl.ds(i * T, T), :], g_ref[...], b_ref[...])
        ln_ref[pl.ds(i * T, T), :] = row
        labr = lab_ref[i, 0, :]

        cls = jax.lax.broadcasted_iota(jnp.int32, (n_cls, T), 0)
        onehot = (cls == labr[None, :]).astype(jnp.float32)
        seg_ref[...] += jax.lax.dot_general(
            onehot, row, (((1,), (0,)), ((), ())),
            preferred_element_type=jnp.float32,
        )
        cnt_ref[...] += jnp.sum(onehot, axis=1)[None, :]
        g = jax.lax.dot_general(
            dic_ref[...], row, (((1,), (1,)), ((), ())),
            preferred_element_type=jnp.float32,
        ) * (1.0 / 768.0)
        dpos_ref[0, 0] += 2.0 * jnp.sum(jnp.where(onehot > 0.0, jnp.exp(g), 0.0))

        lab_all = lab_ref[...]
        row_first = jnp.min(labr)
        jlo = jnp.sum((lab_all < row_first).astype(jnp.int32)) // T

        def body(j, acc):
            col = ln_ref[pl.ds(j * T, T), :]
            s = jax.lax.dot_general(
                row, col, (((1,), (1,)), ((), ())),
                preferred_element_type=jnp.float32,
            ) * (1.0 / 768.0)
            labc = lab_ref[j, 0, :]
            m = labr[:, None] == labc[None, :]
            w = jnp.where(j == i, 1.0, 2.0)
            return acc + w * jnp.sum(jnp.where(m, jnp.exp(s), 0.0))

        acc = jax.lax.fori_loop(jlo, i + 1, body, jnp.float32(0.0))
        posT_ref[0, 0] += acc

    @pl.when(p == nt)
    def _final_phase():
        dic = dic_ref[...]
        diag = jnp.sum(dic * dic, axis=1) * (1.0 / 768.0)
        pos_dic_diag = jnp.sum(jnp.exp(diag))

        cnt = cnt_ref[0, :]
        char_tem = (dic + seg_ref[...]) / (1.0 + cnt)[:, None]
        updated = dic + 0.1 * char_tem
        rowi = jax.lax.broadcasted_iota(jnp.int32, (n_cls, 1), 0)
        new_dic = jnp.where(rowi == 0, dic, updated)
        nd = _ln_rows(new_dic, g_ref[...], b_ref[...])

        sim = jax.lax.dot_general(
            nd, nd, (((1,), (1,)), ((), ())),
            preferred_element_type=jnp.float32,
        ) * (1.0 / 768.0)
        rr = jax.lax.broadcasted_iota(jnp.int32, (n_cls, n_cls), 0)
        cc = jax.lax.broadcasted_iota(jnp.int32, (n_cls, n_cls), 1)
        keep = jnp.logical_and(rr > 0, cc > 0)
        neg = jnp.sum(jnp.where(keep, jnp.exp(sim), 0.0))

        pos = posT_ref[0, 0] + dpos_ref[0, 0] + pos_dic_diag
        o_ref[...] = (-jnp.log(pos / neg)).reshape(1, 1)


def kernel(input_f, target, char_dic, ln1_gamma, ln1_beta):
    B, L, D = input_f.shape
    N = B * L
    n_cls = char_dic.shape[0]
    tokens = input_f.reshape(N, D)
    labels = target.reshape(N).astype(jnp.int32)
    g2 = ln1_gamma.reshape(1, D)
    b2 = ln1_beta.reshape(1, D)

    idx = jnp.arange(N, dtype=jnp.int32)
    skey = jnp.sort(labels * N + idx)
    perm = jnp.bitwise_and(skey, N - 1)
    slab = jax.lax.shift_right_logical(skey, 12)
    stok = _sc_gather(tokens, perm)

    T = 256
    nt = N // T
    lab3 = slab.reshape(nt, 1, T)

    loss = pl.pallas_call(
        functools.partial(_mega_kernel, n_cls, T, nt),
        grid=(nt + 1,),
        in_specs=[
            pl.BlockSpec((N, D), lambda p: (0, 0)),
            pl.BlockSpec((nt, 1, T), lambda p: (0, 0, 0)),
            pl.BlockSpec((n_cls, D), lambda p: (0, 0)),
            pl.BlockSpec((1, D), lambda p: (0, 0)),
            pl.BlockSpec((1, D), lambda p: (0, 0)),
        ],
        out_specs=pl.BlockSpec((1, 1), lambda p: (0, 0)),
        out_shape=jax.ShapeDtypeStruct((1, 1), jnp.float32),
        scratch_shapes=[
            pltpu.VMEM((N, D), jnp.float32),
            pltpu.VMEM((n_cls, D), jnp.float32),
            pltpu.VMEM((1, n_cls), jnp.float32),
            pltpu.SMEM((1, 1), jnp.float32),
            pltpu.SMEM((1, 1), jnp.float32),
        ],
    )(stok, lab3, char_dic, g2, b2)

    return loss.reshape(1)
